# baseline (speedup 1.0000x reference)
import functools
import jax
import jax.numpy as jnp
from jax import lax
from jax.experimental import pallas as pl
from jax.experimental.pallas import tpu as pltpu
from jax.experimental.pallas import tpu_sc as plsc

BS, C, N = 8, 128, 2048
BSH = 4
NH = BS // BSH
BLK = 512
NB = N // BLK


def _prep_body(xt_full_ref, xt_blk_ref, w_ref, br_ref, i2_ref):
    bi = pl.program_id(0)
    xt = xt_full_ref[0]
    xb = xt_blk_ref[0]
    w2 = w_ref[C:, :]

    innerT = lax.dot_general(xt, xb * -2.0, (((0,), (0,)), ((), ())),
                             preferred_element_type=jnp.float32)
    sq_row = jnp.sum(xt * xt, axis=0, keepdims=True)
    sqc = sq_row.T
    score = sqc + innerT

    fiota = lax.broadcasted_iota(jnp.int32, (N, 1), 0).astype(jnp.float32)
    m1 = jnp.min(score, axis=0, keepdims=True)
    score2 = jnp.where(score == m1, 1e30, score)
    m2 = jnp.min(score2, axis=0, keepdims=True)
    a2f = jnp.min(jnp.where(score2 == m2, fiota, float(N)),
                  axis=0, keepdims=True)

    i2_ref[0] = a2f.astype(jnp.int32) + bi * N

    br_ref[...] = lax.dot_general(xb, w2, (((0,), (0,)), ((), ())),
                                  preferred_element_type=jnp.float32)


def _mk_prep(h):
    return pl.pallas_call(
        _prep_body,
        grid=(BSH, NB),
        in_specs=[
            pl.BlockSpec((1, C, N), lambda bi, ii, h=h: (h * BSH + bi, 0, 0)),
            pl.BlockSpec((1, C, BLK),
                         lambda bi, ii, h=h: (h * BSH + bi, 0, ii)),
            pl.BlockSpec((2 * C, C), lambda bi, ii: (0, 0)),
        ],
        out_specs=[
            pl.BlockSpec((BLK, C), lambda bi, ii: (bi * NB + ii, 0)),
            pl.BlockSpec((1, 1, BLK), lambda bi, ii: (bi * NB + ii, 0, 0)),
        ],
        out_shape=[
            jax.ShapeDtypeStruct((BSH * N, C), jnp.float32),
            jax.ShapeDtypeStruct((BSH * NB, 1, BLK), jnp.int32),
        ],
    )


_prep_calls = [_mk_prep(h) for h in range(NH)]


def _combine_compute(xt_ref, w_ref, b2_ref, m_ref, o_ref, bo):
    xt = xt_ref[0]
    w1 = w_ref[:C, :]
    w2 = w_ref[C:, :]
    at = lax.dot_general(w1 - w2, xt, (((0,), (0,)), ((), ())),
                         preferred_element_type=jnp.float32)
    bt = lax.dot_general(w2, xt, (((0,), (0,)), ((), ())),
                         preferred_element_type=jnp.float32)
    mt = m_ref[0].T
    h = (at + b2_ref[...]) + jnp.maximum(bt, mt)
    o_ref[bo] = jnp.where(h > 0, h, 0.2 * h)


def _combine_body0(xt_ref, w_ref, b2_ref, m_ref, o_ref):
    _combine_compute(xt_ref, w_ref, b2_ref, m_ref, o_ref, 0)


def _combine_body1(acc_ref, xt_ref, w_ref, b2_ref, m_ref, o_ref):
    del acc_ref
    _combine_compute(xt_ref, w_ref, b2_ref, m_ref, o_ref, 0)


def _mk_combine(h):
    xt_spec = pl.BlockSpec((1, C, N), lambda bi, h=h: (h * BSH + bi, 0, 0))
    common = [
        xt_spec,
        pl.BlockSpec((2 * C, C), lambda bi: (0, 0)),
        pl.BlockSpec((C, 1), lambda bi: (0, 0)),
        pl.BlockSpec((1, N, C), lambda bi: (bi, 0, 0)),
    ]
    out_spec = pl.BlockSpec((1, C, N), lambda bi, h=h: (h * BSH + bi, 0, 0))
    out_shape = jax.ShapeDtypeStruct((BS, C, N), jnp.float32)
    if h == 0:
        return pl.pallas_call(
            _combine_body0, grid=(BSH,), in_specs=common,
            out_specs=out_spec, out_shape=out_shape,
        )
    return pl.pallas_call(
        _combine_body1, grid=(BSH,),
        in_specs=[pl.BlockSpec(memory_space=pl.ANY)] + common,
        out_specs=out_spec, out_shape=out_shape,
        input_output_aliases={0: 0},
    )


_combine_calls = [_mk_combine(h) for h in range(NH)]


_NC, _NS, _L = 2, 16, 16
NW = _NC * _NS
ROWS_PER_W = (BSH * N) // NW
CB = 256
NCHUNK = ROWS_PER_W // CB


@functools.cache
def _make_sc_gather():
    mesh = plsc.VectorSubcoreMesh(core_axis_name="c", subcore_axis_name="s")

    @functools.partial(
        pl.kernel,
        mesh=mesh,
        out_type=jax.ShapeDtypeStruct((BSH * N, C), jnp.float32),
        scratch_types=[
            pltpu.VMEM((CB,), jnp.int32),
            pltpu.VMEM((CB,), jnp.int32),
            pltpu.VMEM((CB, C), jnp.float32),
            pltpu.VMEM((CB, C), jnp.float32),
            pltpu.SemaphoreType.DMA,
            pltpu.SemaphoreType.DMA,
        ],
    )
    def sc_gather(br_hbm, i2_hbm, out_hbm, iva, ivb, ga, gb, sa, sb):
        wid = lax.axis_index("s") * _NC + lax.axis_index("c")
        base = wid * ROWS_PER_W
        idx_v = (iva, ivb)
        g_v = (ga, gb)
        sem = (sa, sb)
        copies = [None, None]
        pltpu.sync_copy(i2_hbm.at[pl.ds(base, CB)], iva)
        copies[0] = pltpu.async_copy(br_hbm.at[iva], ga, sa)
        for ci in range(NCHUNK):
            cur = ci % 2
            nxt = (ci + 1) % 2
            if ci + 1 < NCHUNK:
                off_n = base + (ci + 1) * CB
                pltpu.sync_copy(i2_hbm.at[pl.ds(off_n, CB)], idx_v[nxt])
                copies[nxt] = pltpu.async_copy(
                    br_hbm.at[idx_v[nxt]], g_v[nxt], sem[nxt])
            copies[cur].wait()
            pltpu.sync_copy(g_v[cur], out_hbm.at[pl.ds(base + ci * CB, CB)])

    return sc_gather


def kernel(sparse_fea, W, b):
    b2 = b.reshape(C, 1)
    sc = _make_sc_gather()
    ms = []
    for h in range(NH):
        brows, i2 = _prep_calls[h](sparse_fea, sparse_fea, W)
        ms.append(sc(brows, i2.reshape(BSH * N)))
    out = _combine_calls[0](sparse_fea, W, b2, ms[0].reshape(BSH, N, C))
    for h in range(1, NH):
        out = _combine_calls[h](out, sparse_fea, W, b2,
                                ms[h].reshape(BSH, N, C))
    return out


# speedup vs baseline: 1.0168x; 1.0168x over previous
import functools
import jax
import jax.numpy as jnp
from jax import lax
from jax.experimental import pallas as pl
from jax.experimental.pallas import tpu as pltpu
from jax.experimental.pallas import tpu_sc as plsc

BS, C, N = 8, 128, 2048
BSH = 4
NH = BS // BSH
BLK = 512
NB = N // BLK


def _prep_body(xt_full_ref, xt_blk_ref, w_ref, iota_ref, br_ref, i2_ref):
    bi = pl.program_id(0)
    xt = xt_full_ref[0]
    xb = xt_blk_ref[0]
    w2 = w_ref[C:, :]

    innerT = lax.dot_general(xt, xb * -2.0, (((0,), (0,)), ((), ())),
                             preferred_element_type=jnp.float32)
    sq_row = jnp.sum(xt * xt, axis=0, keepdims=True)
    sqc = sq_row.T
    score = sqc + innerT

    fiota = iota_ref[...]
    m1 = jnp.min(score, axis=0, keepdims=True)
    score2 = jnp.where(score == m1, 1e30, score)
    m2 = jnp.min(score2, axis=0, keepdims=True)
    a2f = jnp.min(jnp.where(score2 == m2, fiota, float(N)),
                  axis=0, keepdims=True)

    i2_ref[0] = a2f.astype(jnp.int32) + bi * N

    br_ref[...] = lax.dot_general(xb, w2, (((0,), (0,)), ((), ())),
                                  preferred_element_type=jnp.float32)


def _mk_prep(h):
    return pl.pallas_call(
        _prep_body,
        grid=(BSH, NB),
        in_specs=[
            pl.BlockSpec((1, C, N), lambda bi, ii, h=h: (h * BSH + bi, 0, 0)),
            pl.BlockSpec((1, C, BLK),
                         lambda bi, ii, h=h: (h * BSH + bi, 0, ii)),
            pl.BlockSpec((2 * C, C), lambda bi, ii: (0, 0)),
            pl.BlockSpec((N, 1), lambda bi, ii: (0, 0)),
        ],
        out_specs=[
            pl.BlockSpec((BLK, C), lambda bi, ii: (bi * NB + ii, 0)),
            pl.BlockSpec((1, 1, BLK), lambda bi, ii: (bi * NB + ii, 0, 0)),
        ],
        out_shape=[
            jax.ShapeDtypeStruct((BSH * N, C), jnp.float32),
            jax.ShapeDtypeStruct((BSH * NB, 1, BLK), jnp.int32),
        ],
    )


_prep_calls = [_mk_prep(h) for h in range(NH)]


def _combine_compute(xt_ref, w_ref, b2_ref, m_ref, o_ref, bo):
    xt = xt_ref[0]
    w1 = w_ref[:C, :]
    w2 = w_ref[C:, :]
    at = lax.dot_general(w1 - w2, xt, (((0,), (0,)), ((), ())),
                         preferred_element_type=jnp.float32)
    bt = lax.dot_general(w2, xt, (((0,), (0,)), ((), ())),
                         preferred_element_type=jnp.float32)
    mt = m_ref[0].T
    h = (at + b2_ref[...]) + jnp.maximum(bt, mt)
    o_ref[bo] = jnp.where(h > 0, h, 0.2 * h)


def _combine_body0(xt_ref, w_ref, b2_ref, m_ref, o_ref):
    _combine_compute(xt_ref, w_ref, b2_ref, m_ref, o_ref, 0)


def _combine_body1(acc_ref, xt_ref, w_ref, b2_ref, m_ref, o_ref):
    del acc_ref
    _combine_compute(xt_ref, w_ref, b2_ref, m_ref, o_ref, 0)


def _mk_combine(h):
    xt_spec = pl.BlockSpec((1, C, N), lambda bi, h=h: (h * BSH + bi, 0, 0))
    common = [
        xt_spec,
        pl.BlockSpec((2 * C, C), lambda bi: (0, 0)),
        pl.BlockSpec((C, 1), lambda bi: (0, 0)),
        pl.BlockSpec((1, N, C), lambda bi: (bi, 0, 0)),
    ]
    out_spec = pl.BlockSpec((1, C, N), lambda bi, h=h: (h * BSH + bi, 0, 0))
    out_shape = jax.ShapeDtypeStruct((BS, C, N), jnp.float32)
    if h == 0:
        return pl.pallas_call(
            _combine_body0, grid=(BSH,), in_specs=common,
            out_specs=out_spec, out_shape=out_shape,
        )
    return pl.pallas_call(
        _combine_body1, grid=(BSH,),
        in_specs=[pl.BlockSpec(memory_space=pl.ANY)] + common,
        out_specs=out_spec, out_shape=out_shape,
        input_output_aliases={0: 0},
    )


_combine_calls = [_mk_combine(h) for h in range(NH)]


_NC, _NS, _L = 2, 16, 16
NW = _NC * _NS
ROWS_PER_W = (BSH * N) // NW
CB = 256
NCHUNK = ROWS_PER_W // CB


@functools.cache
def _make_sc_gather():
    mesh = plsc.VectorSubcoreMesh(core_axis_name="c", subcore_axis_name="s")

    @functools.partial(
        pl.kernel,
        mesh=mesh,
        out_type=jax.ShapeDtypeStruct((BSH * N, C), jnp.float32),
        scratch_types=[
            pltpu.VMEM((CB,), jnp.int32),
            pltpu.VMEM((CB,), jnp.int32),
            pltpu.VMEM((CB, C), jnp.float32),
            pltpu.VMEM((CB, C), jnp.float32),
            pltpu.SemaphoreType.DMA,
            pltpu.SemaphoreType.DMA,
        ],
    )
    def sc_gather(br_hbm, i2_hbm, out_hbm, iva, ivb, ga, gb, sa, sb):
        wid = lax.axis_index("s") * _NC + lax.axis_index("c")
        base = wid * ROWS_PER_W
        idx_v = (iva, ivb)
        g_v = (ga, gb)
        sem = (sa, sb)
        copies = [None, None]
        pltpu.sync_copy(i2_hbm.at[pl.ds(base, CB)], iva)
        copies[0] = pltpu.async_copy(br_hbm.at[iva], ga, sa)
        for ci in range(NCHUNK):
            cur = ci % 2
            nxt = (ci + 1) % 2
            if ci + 1 < NCHUNK:
                off_n = base + (ci + 1) * CB
                pltpu.sync_copy(i2_hbm.at[pl.ds(off_n, CB)], idx_v[nxt])
                copies[nxt] = pltpu.async_copy(
                    br_hbm.at[idx_v[nxt]], g_v[nxt], sem[nxt])
            copies[cur].wait()
            pltpu.sync_copy(g_v[cur], out_hbm.at[pl.ds(base + ci * CB, CB)])

    return sc_gather


def kernel(sparse_fea, W, b):
    iota_col = jnp.arange(N, dtype=jnp.float32).reshape(N, 1)
    b2 = b.reshape(C, 1)
    sc = _make_sc_gather()
    ms = []
    for h in range(NH):
        brows, i2 = _prep_calls[h](sparse_fea, sparse_fea, W, iota_col)
        ms.append(sc(brows, i2.reshape(BSH * N)))
    out = _combine_calls[0](sparse_fea, W, b2, ms[0].reshape(BSH, N, C))
    for h in range(1, NH):
        out = _combine_calls[h](out, sparse_fea, W, b2,
                                ms[h].reshape(BSH, N, C))
    return out


# speedup vs baseline: 1.1102x; 1.0919x over previous
import functools
import jax
import jax.numpy as jnp
from jax import lax
from jax.experimental import pallas as pl
from jax.experimental.pallas import tpu as pltpu
from jax.experimental.pallas import tpu_sc as plsc

BS, C, N = 8, 128, 2048
BSH = 4
NH = BS // BSH
BLK = 1024
NB = N // BLK


def _prep_body(xt_full_ref, xt_blk_ref, w_ref, iota_ref, br_ref, i2_ref):
    bi = pl.program_id(0)
    xt = xt_full_ref[0]
    xb = xt_blk_ref[0]
    w2 = w_ref[C:, :]

    innerT = lax.dot_general(xt, xb * -2.0, (((0,), (0,)), ((), ())),
                             preferred_element_type=jnp.float32)
    sq_row = jnp.sum(xt * xt, axis=0, keepdims=True)
    sqc = sq_row.T
    score = sqc + innerT

    fiota = iota_ref[...]
    m1 = jnp.min(score, axis=0, keepdims=True)
    score2 = jnp.where(score == m1, 1e30, score)
    m2 = jnp.min(score2, axis=0, keepdims=True)
    a2f = jnp.min(jnp.where(score2 == m2, fiota, float(N)),
                  axis=0, keepdims=True)

    i2_ref[0] = a2f.astype(jnp.int32) + bi * N

    br_ref[...] = lax.dot_general(xb, w2, (((0,), (0,)), ((), ())),
                                  preferred_element_type=jnp.float32)


def _mk_prep(h):
    return pl.pallas_call(
        _prep_body,
        grid=(BSH, NB),
        in_specs=[
            pl.BlockSpec((1, C, N), lambda bi, ii, h=h: (h * BSH + bi, 0, 0)),
            pl.BlockSpec((1, C, BLK),
                         lambda bi, ii, h=h: (h * BSH + bi, 0, ii)),
            pl.BlockSpec((2 * C, C), lambda bi, ii: (0, 0)),
            pl.BlockSpec((N, 1), lambda bi, ii: (0, 0)),
        ],
        out_specs=[
            pl.BlockSpec((BLK, C), lambda bi, ii: (bi * NB + ii, 0)),
            pl.BlockSpec((1, 1, BLK), lambda bi, ii: (bi * NB + ii, 0, 0)),
        ],
        out_shape=[
            jax.ShapeDtypeStruct((BSH * N, C), jnp.float32),
            jax.ShapeDtypeStruct((BSH * NB, 1, BLK), jnp.int32),
        ],
    )


_prep_calls = [_mk_prep(h) for h in range(NH)]


def _combine_compute(xt_ref, w_ref, b2_ref, m_ref, o_ref, bo):
    xt = xt_ref[0]
    w1 = w_ref[:C, :]
    w2 = w_ref[C:, :]
    at = lax.dot_general(w1 - w2, xt, (((0,), (0,)), ((), ())),
                         preferred_element_type=jnp.float32)
    bt = lax.dot_general(w2, xt, (((0,), (0,)), ((), ())),
                         preferred_element_type=jnp.float32)
    mt = m_ref[0].T
    h = (at + b2_ref[...]) + jnp.maximum(bt, mt)
    o_ref[bo] = jnp.where(h > 0, h, 0.2 * h)


def _combine_body0(xt_ref, w_ref, b2_ref, m_ref, o_ref):
    _combine_compute(xt_ref, w_ref, b2_ref, m_ref, o_ref, 0)


def _combine_body1(acc_ref, xt_ref, w_ref, b2_ref, m_ref, o_ref):
    del acc_ref
    _combine_compute(xt_ref, w_ref, b2_ref, m_ref, o_ref, 0)


def _mk_combine(h):
    xt_spec = pl.BlockSpec((1, C, N), lambda bi, h=h: (h * BSH + bi, 0, 0))
    common = [
        xt_spec,
        pl.BlockSpec((2 * C, C), lambda bi: (0, 0)),
        pl.BlockSpec((C, 1), lambda bi: (0, 0)),
        pl.BlockSpec((1, N, C), lambda bi: (bi, 0, 0)),
    ]
    out_spec = pl.BlockSpec((1, C, N), lambda bi, h=h: (h * BSH + bi, 0, 0))
    out_shape = jax.ShapeDtypeStruct((BS, C, N), jnp.float32)
    if h == 0:
        return pl.pallas_call(
            _combine_body0, grid=(BSH,), in_specs=common,
            out_specs=out_spec, out_shape=out_shape,
        )
    return pl.pallas_call(
        _combine_body1, grid=(BSH,),
        in_specs=[pl.BlockSpec(memory_space=pl.ANY)] + common,
        out_specs=out_spec, out_shape=out_shape,
        input_output_aliases={0: 0},
    )


_combine_calls = [_mk_combine(h) for h in range(NH)]


_NC, _NS, _L = 2, 16, 16
NW = _NC * _NS
ROWS_PER_W = (BSH * N) // NW
CB = 256
NCHUNK = ROWS_PER_W // CB


@functools.cache
def _make_sc_gather():
    mesh = plsc.VectorSubcoreMesh(core_axis_name="c", subcore_axis_name="s")

    @functools.partial(
        pl.kernel,
        mesh=mesh,
        out_type=jax.ShapeDtypeStruct((BSH * N, C), jnp.float32),
        scratch_types=[
            pltpu.VMEM((CB,), jnp.int32),
            pltpu.VMEM((CB,), jnp.int32),
            pltpu.VMEM((CB, C), jnp.float32),
            pltpu.VMEM((CB, C), jnp.float32),
            pltpu.SemaphoreType.DMA,
            pltpu.SemaphoreType.DMA,
        ],
    )
    def sc_gather(br_hbm, i2_hbm, out_hbm, iva, ivb, ga, gb, sa, sb):
        wid = lax.axis_index("s") * _NC + lax.axis_index("c")
        base = wid * ROWS_PER_W
        idx_v = (iva, ivb)
        g_v = (ga, gb)
        sem = (sa, sb)
        copies = [None, None]
        pltpu.sync_copy(i2_hbm.at[pl.ds(base, CB)], iva)
        copies[0] = pltpu.async_copy(br_hbm.at[iva], ga, sa)
        for ci in range(NCHUNK):
            cur = ci % 2
            nxt = (ci + 1) % 2
            if ci + 1 < NCHUNK:
                off_n = base + (ci + 1) * CB
                pltpu.sync_copy(i2_hbm.at[pl.ds(off_n, CB)], idx_v[nxt])
                copies[nxt] = pltpu.async_copy(
                    br_hbm.at[idx_v[nxt]], g_v[nxt], sem[nxt])
            copies[cur].wait()
            pltpu.sync_copy(g_v[cur], out_hbm.at[pl.ds(base + ci * CB, CB)])

    return sc_gather


def kernel(sparse_fea, W, b):
    iota_col = jnp.arange(N, dtype=jnp.float32).reshape(N, 1)
    b2 = b.reshape(C, 1)
    sc = _make_sc_gather()
    ms = []
    for h in range(NH):
        brows, i2 = _prep_calls[h](sparse_fea, sparse_fea, W, iota_col)
        ms.append(sc(brows, i2.reshape(BSH * N)))
    out = _combine_calls[0](sparse_fea, W, b2, ms[0].reshape(BSH, N, C))
    for h in range(1, NH):
        out = _combine_calls[h](out, sparse_fea, W, b2,
                                ms[h].reshape(BSH, N, C))
    return out


# speedup vs baseline: 1.1783x; 1.0613x over previous
import functools
import jax
import jax.numpy as jnp
from jax import lax
from jax.experimental import pallas as pl
from jax.experimental.pallas import tpu as pltpu
from jax.experimental.pallas import tpu_sc as plsc

BS, C, N = 8, 128, 2048
BSH = 4
NH = BS // BSH
BLK = 2048
NB = N // BLK


def _prep_body(xt_full_ref, xt_blk_ref, w_ref, iota_ref, br_ref, i2_ref):
    bi = pl.program_id(0)
    xt = xt_full_ref[0]
    xb = xt_blk_ref[0]
    w2 = w_ref[C:, :]

    innerT = lax.dot_general(xt, xb * -2.0, (((0,), (0,)), ((), ())),
                             preferred_element_type=jnp.float32)
    sq_row = jnp.sum(xt * xt, axis=0, keepdims=True)
    sqc = sq_row.T
    score = sqc + innerT

    fiota = iota_ref[...]
    m1 = jnp.min(score, axis=0, keepdims=True)
    score2 = jnp.where(score == m1, 1e30, score)
    m2 = jnp.min(score2, axis=0, keepdims=True)
    a2f = jnp.min(jnp.where(score2 == m2, fiota, float(N)),
                  axis=0, keepdims=True)

    i2_ref[0] = a2f.astype(jnp.int32) + bi * N

    br_ref[...] = lax.dot_general(xb, w2, (((0,), (0,)), ((), ())),
                                  preferred_element_type=jnp.float32)


def _mk_prep(h):
    return pl.pallas_call(
        _prep_body,
        grid=(BSH, NB),
        in_specs=[
            pl.BlockSpec((1, C, N), lambda bi, ii, h=h: (h * BSH + bi, 0, 0)),
            pl.BlockSpec((1, C, BLK),
                         lambda bi, ii, h=h: (h * BSH + bi, 0, ii)),
            pl.BlockSpec((2 * C, C), lambda bi, ii: (0, 0)),
            pl.BlockSpec((N, 1), lambda bi, ii: (0, 0)),
        ],
        out_specs=[
            pl.BlockSpec((BLK, C), lambda bi, ii: (bi * NB + ii, 0)),
            pl.BlockSpec((1, 1, BLK), lambda bi, ii: (bi * NB + ii, 0, 0)),
        ],
        out_shape=[
            jax.ShapeDtypeStruct((BSH * N, C), jnp.float32),
            jax.ShapeDtypeStruct((BSH * NB, 1, BLK), jnp.int32),
        ],
    )


_prep_calls = [_mk_prep(h) for h in range(NH)]


def _combine_compute(xt_ref, w_ref, b2_ref, m_ref, o_ref, bo):
    xt = xt_ref[0]
    w1 = w_ref[:C, :]
    w2 = w_ref[C:, :]
    at = lax.dot_general(w1 - w2, xt, (((0,), (0,)), ((), ())),
                         preferred_element_type=jnp.float32)
    bt = lax.dot_general(w2, xt, (((0,), (0,)), ((), ())),
                         preferred_element_type=jnp.float32)
    mt = m_ref[0].T
    h = (at + b2_ref[...]) + jnp.maximum(bt, mt)
    o_ref[bo] = jnp.where(h > 0, h, 0.2 * h)


def _combine_body0(xt_ref, w_ref, b2_ref, m_ref, o_ref):
    _combine_compute(xt_ref, w_ref, b2_ref, m_ref, o_ref, 0)


def _combine_body1(acc_ref, xt_ref, w_ref, b2_ref, m_ref, o_ref):
    del acc_ref
    _combine_compute(xt_ref, w_ref, b2_ref, m_ref, o_ref, 0)


def _mk_combine(h):
    xt_spec = pl.BlockSpec((1, C, N), lambda bi, h=h: (h * BSH + bi, 0, 0))
    common = [
        xt_spec,
        pl.BlockSpec((2 * C, C), lambda bi: (0, 0)),
        pl.BlockSpec((C, 1), lambda bi: (0, 0)),
        pl.BlockSpec((1, N, C), lambda bi: (bi, 0, 0)),
    ]
    out_spec = pl.BlockSpec((1, C, N), lambda bi, h=h: (h * BSH + bi, 0, 0))
    out_shape = jax.ShapeDtypeStruct((BS, C, N), jnp.float32)
    if h == 0:
        return pl.pallas_call(
            _combine_body0, grid=(BSH,), in_specs=common,
            out_specs=out_spec, out_shape=out_shape,
        )
    return pl.pallas_call(
        _combine_body1, grid=(BSH,),
        in_specs=[pl.BlockSpec(memory_space=pl.ANY)] + common,
        out_specs=out_spec, out_shape=out_shape,
        input_output_aliases={0: 0},
    )


_combine_calls = [_mk_combine(h) for h in range(NH)]


_NC, _NS, _L = 2, 16, 16
NW = _NC * _NS
ROWS_PER_W = (BSH * N) // NW
CB = 256
NCHUNK = ROWS_PER_W // CB


@functools.cache
def _make_sc_gather():
    mesh = plsc.VectorSubcoreMesh(core_axis_name="c", subcore_axis_name="s")

    @functools.partial(
        pl.kernel,
        mesh=mesh,
        out_type=jax.ShapeDtypeStruct((BSH * N, C), jnp.float32),
        scratch_types=[
            pltpu.VMEM((CB,), jnp.int32),
            pltpu.VMEM((CB,), jnp.int32),
            pltpu.VMEM((CB, C), jnp.float32),
            pltpu.VMEM((CB, C), jnp.float32),
            pltpu.SemaphoreType.DMA,
            pltpu.SemaphoreType.DMA,
        ],
    )
    def sc_gather(br_hbm, i2_hbm, out_hbm, iva, ivb, ga, gb, sa, sb):
        wid = lax.axis_index("s") * _NC + lax.axis_index("c")
        base = wid * ROWS_PER_W
        idx_v = (iva, ivb)
        g_v = (ga, gb)
        sem = (sa, sb)
        copies = [None, None]
        pltpu.sync_copy(i2_hbm.at[pl.ds(base, CB)], iva)
        copies[0] = pltpu.async_copy(br_hbm.at[iva], ga, sa)
        for ci in range(NCHUNK):
            cur = ci % 2
            nxt = (ci + 1) % 2
            if ci + 1 < NCHUNK:
                off_n = base + (ci + 1) * CB
                pltpu.sync_copy(i2_hbm.at[pl.ds(off_n, CB)], idx_v[nxt])
                copies[nxt] = pltpu.async_copy(
                    br_hbm.at[idx_v[nxt]], g_v[nxt], sem[nxt])
            copies[cur].wait()
            pltpu.sync_copy(g_v[cur], out_hbm.at[pl.ds(base + ci * CB, CB)])

    return sc_gather


def kernel(sparse_fea, W, b):
    iota_col = jnp.arange(N, dtype=jnp.float32).reshape(N, 1)
    b2 = b.reshape(C, 1)
    sc = _make_sc_gather()
    ms = []
    for h in range(NH):
        brows, i2 = _prep_calls[h](sparse_fea, sparse_fea, W, iota_col)
        ms.append(sc(brows, i2.reshape(BSH * N)))
    out = _combine_calls[0](sparse_fea, W, b2, ms[0].reshape(BSH, N, C))
    for h in range(1, NH):
        out = _combine_calls[h](out, sparse_fea, W, b2,
                                ms[h].reshape(BSH, N, C))
    return out
